# baseline (speedup 1.0000x reference)
import functools

import jax
import jax.numpy as jnp
from jax import lax
from jax.experimental import pallas as pl
from jax.experimental.pallas import tpu as pltpu
from jax.experimental.pallas import tpu_sc as plsc

N_USERS = 1000000
N_ITEMS = 100000
K = 32
B = 16384
MU = 3.5

_INFO = plsc.get_sparse_core_info()
NC, NS, L = _INFO.num_cores, _INFO.num_subcores, _INFO.num_lanes
NW = NC * NS
BPW = B // NW
GROUPS = BPW // L


def _lfm_kernel(uidx_hbm, iidx_hbm, p_hbm, q_hbm, bu_hbm, bi_hbm, out_hbm,
                uidx_v, iidx_v, p_v, q_v, bu_v, bi_v, o_v, sem):
    wid = lax.axis_index("s") * NC + lax.axis_index("c")
    base = wid * BPW

    pltpu.sync_copy(uidx_hbm.at[pl.ds(base, BPW)], uidx_v)
    pltpu.sync_copy(iidx_hbm.at[pl.ds(base, BPW)], iidx_v)

    cps = [
        pltpu.async_copy(p_hbm.at[uidx_v], p_v, sem),
        pltpu.async_copy(q_hbm.at[iidx_v], q_v, sem),
        pltpu.async_copy(bu_hbm.at[uidx_v], bu_v, sem),
        pltpu.async_copy(bi_hbm.at[iidx_v], bi_v, sem),
    ]
    for cp in cps:
        cp.wait()

    lane = lax.iota(jnp.int32, L)

    def group(g, carry):
        rows = g * L + lane
        acc = MU + bu_v[pl.ds(g * L, L)] + bi_v[pl.ds(g * L, L)]
        for k in range(K):
            col = jnp.full((L,), k, jnp.int32)
            pk = plsc.load_gather(p_v, [rows, col])
            qk = plsc.load_gather(q_v, [rows, col])
            acc = acc + pk * qk
        o_v[pl.ds(g * L, L)] = acc
        return carry

    lax.fori_loop(0, GROUPS, group, 0)
    pltpu.sync_copy(o_v, out_hbm.at[pl.ds(base, BPW)])


@jax.jit
def kernel(user_idx, item_idx, P, Q, b_u, b_i):
    mesh = plsc.VectorSubcoreMesh(core_axis_name="c", subcore_axis_name="s")
    run = functools.partial(
        pl.kernel,
        mesh=mesh,
        out_type=jax.ShapeDtypeStruct((B,), jnp.float32),
        scratch_types=[
            pltpu.VMEM((BPW,), jnp.int32),
            pltpu.VMEM((BPW,), jnp.int32),
            pltpu.VMEM((BPW, K), jnp.float32),
            pltpu.VMEM((BPW, K), jnp.float32),
            pltpu.VMEM((BPW,), jnp.float32),
            pltpu.VMEM((BPW,), jnp.float32),
            pltpu.VMEM((BPW,), jnp.float32),
            pltpu.SemaphoreType.DMA,
        ],
        compiler_params=pltpu.CompilerParams(
            needs_layout_passes=False, use_tc_tiling_on_sc=False),
    )(_lfm_kernel)
    return run(user_idx, item_idx, P, Q, b_u.reshape(-1), b_i.reshape(-1))
